# baseline (speedup 1.0000x reference)
import functools
import math

import jax
import jax.numpy as jnp
from jax import lax
from jax.experimental import pallas as pl
from jax.experimental.pallas import tpu as pltpu
from jax.experimental.pallas import tpu_sc as plsc

D_MODEL = 64
SCALE = math.sqrt(D_MODEL)
HALF = 32

_info = plsc.get_sparse_core_info()
_NC, _NS, _L = _info.num_cores, _info.num_subcores, _info.num_lanes
_NW = _NC * _NS


@functools.partial(jax.jit, static_argnames=("n_half",))
def _emb_lookup(idx2, table2, n_half):
    h_per_w = n_half // _NW
    ch = 128
    n_ch = h_per_w // ch
    mesh = plsc.VectorSubcoreMesh(core_axis_name="c", subcore_axis_name="s")

    @functools.partial(
        pl.kernel,
        mesh=mesh,
        out_type=jax.ShapeDtypeStruct((n_half, HALF), jnp.float32),
        scratch_types=[
            pltpu.VMEM((n_ch, ch), jnp.int32),
            pltpu.VMEM((h_per_w, HALF), jnp.float32),
            pltpu.SemaphoreType.DMA,
        ],
        compiler_params=pltpu.CompilerParams(use_tc_tiling_on_sc=False),
    )
    def k(idx_hbm, tab_hbm, out_hbm, idx_v, rows_v, sem):
        wid = lax.axis_index("s") * _NC + lax.axis_index("c")
        base = wid * h_per_w
        pltpu.sync_copy(idx_hbm.at[wid], idx_v)
        copies = []
        for c in range(n_ch):
            copies.append(
                pltpu.async_copy(
                    tab_hbm.at[idx_v.at[c]],
                    rows_v.at[pl.ds(c * ch, ch)],
                    sem,
                )
            )
        for cp in copies:
            cp.wait()

        def scale_row(i, carry):
            for kk in range(HALF // _L):
                sl = pl.ds(kk * _L, _L)
                rows_v[i, sl] = rows_v[i, sl] * SCALE
            return carry

        lax.fori_loop(0, h_per_w, scale_row, 0)
        pltpu.sync_copy(rows_v, out_hbm.at[pl.ds(base, h_per_w)])

    return k(idx2, table2)


def kernel(x, weight):
    b0, b1 = x.shape
    b_total = b0 * b1
    n_half = 2 * b_total
    r = x.astype(jnp.int32).reshape(-1)
    idx2 = jnp.stack([r * 2, r * 2 + 1], axis=-1).reshape(_NW, -1, 128)
    table2 = weight.reshape(weight.shape[0] * 2, HALF)
    out = _emb_lookup(idx2, table2, n_half)
    return out.reshape(b0, b1, D_MODEL)


# speedup vs baseline: 1.0018x; 1.0018x over previous
import functools
import math

import jax
import jax.numpy as jnp
from jax import lax
from jax.experimental import pallas as pl
from jax.experimental.pallas import tpu as pltpu
from jax.experimental.pallas import tpu_sc as plsc

D_MODEL = 64
SCALE = math.sqrt(D_MODEL)
VROW = 128

_info = plsc.get_sparse_core_info()
_NC, _NS, _L = _info.num_cores, _info.num_subcores, _info.num_lanes
_NW = _NC * _NS


@functools.partial(jax.jit, static_argnames=("b_total", "d"))
def _emb_lookup(x3, table2, b_total, d):
    b_per_w = b_total // _NW
    ich = 64
    n_ich = b_per_w // ich
    och = 128
    n_och = b_per_w // och
    mesh = plsc.VectorSubcoreMesh(core_axis_name="c", subcore_axis_name="s")

    @functools.partial(
        pl.kernel,
        mesh=mesh,
        out_type=jax.ShapeDtypeStruct((b_total, d), jnp.float32),
        scratch_types=[
            pltpu.VMEM((8, ich), jnp.int32),
            pltpu.VMEM((n_ich, ich), jnp.int32),
            pltpu.VMEM((b_per_w, VROW), jnp.float32),
            pltpu.VMEM((och, d), jnp.float32),
            pltpu.SemaphoreType.DMA,
        ],
    )
    def k(idx_hbm, tab_hbm, out_hbm, idx_v, grp_v, buf_v, stage_v, sem):
        wid = lax.axis_index("s") * _NC + lax.axis_index("c")
        base = wid * b_per_w
        pltpu.sync_copy(idx_hbm.at[wid], idx_v)

        for j in range(8):
            for kk in range(ich // _L):
                grp_v[j, pl.ds(kk * _L, _L)] = idx_v[j, pl.ds(kk * _L, _L)] >> 1

        copies = []
        for c in range(n_ich):
            copies.append(
                pltpu.async_copy(
                    tab_hbm.at[grp_v.at[c]],
                    buf_v.at[pl.ds(c * ich, ich)],
                    sem,
                )
            )
        for cp in copies:
            cp.wait()

        for c in range(n_och):
            def sub_body(b, carry):
                j = (c * och + b * _L) // ich
                col = (c * och + b * _L) % ich
                ovec = idx_v[j, pl.ds(col, _L)] & 1
                for s in range(_L):
                    flat = c * och + b * _L + s
                    half = ovec[s] * d
                    srow = b * _L + s
                    for kk in range(d // _L):
                        sl = pl.ds(kk * _L, _L)
                        stage_v[srow, sl] = (
                            buf_v[flat, pl.ds(half + kk * _L, _L)] * SCALE
                        )
                return carry

            lax.fori_loop(0, och // _L, sub_body, 0)
            pltpu.sync_copy(stage_v, out_hbm.at[pl.ds(base + c * och, och)])

    return k(x3, table2)


def kernel(x, weight):
    b0, b1 = x.shape
    b_total = b0 * b1
    b_per_w = b_total // _NW
    x3 = x.astype(jnp.int32).reshape(_NW, 8, b_per_w // 8)
    table2 = weight.reshape(weight.shape[0] // 2, VROW)
    out = _emb_lookup(x3, table2, b_total, D_MODEL)
    return out.reshape(b0, b1, D_MODEL)


# speedup vs baseline: 2.3079x; 2.3038x over previous
import functools
import math

import jax
import jax.numpy as jnp
from jax import lax
from jax.experimental import pallas as pl
from jax.experimental.pallas import tpu as pltpu
from jax.experimental.pallas import tpu_sc as plsc

D_MODEL = 64
SCALE = math.sqrt(D_MODEL)
RPG = 8

_info = plsc.get_sparse_core_info()
_NC, _NS, _L = _info.num_cores, _info.num_subcores, _info.num_lanes
_NW = _NC * _NS


@functools.partial(jax.jit, static_argnames=("b_total", "d"))
def _emb_lookup(x3, table3, b_total, d):
    b_per_w = b_total // _NW
    n_batch = b_per_w // _L
    och = 128
    mesh = plsc.VectorSubcoreMesh(core_axis_name="c", subcore_axis_name="s")

    @functools.partial(
        pl.kernel,
        mesh=mesh,
        out_type=jax.ShapeDtypeStruct((b_total, d), jnp.float32),
        scratch_types=[
            pltpu.VMEM((RPG, b_per_w // RPG), jnp.int32),
            pltpu.VMEM((2 * _L, RPG, d), jnp.float32),
            pltpu.VMEM((och, d), jnp.float32),
            pltpu.SemaphoreType.DMA,
            pltpu.SemaphoreType.DMA,
        ],
    )
    def k(idx_hbm, tab_hbm, out_hbm, idx_v, buf_v, stage_v, sem_a, sem_b):
        wid = lax.axis_index("s") * _NC + lax.axis_index("c")
        base = wid * b_per_w
        pltpu.sync_copy(idx_hbm.at[wid], idx_v)
        ncol = b_per_w // RPG

        def batch_vec(bb):
            j = (bb * _L) // ncol
            col = (bb * _L) % ncol
            return idx_v[j, pl.ds(col, _L)]

        def fire(bb, sem):
            gv = batch_vec(bb) >> 3
            half = (bb & 1) * _L
            for s in range(_L):
                pltpu.async_copy(tab_hbm.at[gv[s]], buf_v.at[half + s], sem)

        def drain(sem):
            for _ in range(_L):
                pltpu.make_async_copy(tab_hbm.at[0], buf_v.at[0], sem).wait()

        def select(bb):
            ov = batch_vec(bb) & 7
            half = (bb & 1) * _L
            for s in range(_L):
                srow = ((bb * _L) % och) + s
                for kk in range(d // _L):
                    sl = pl.ds(kk * _L, _L)
                    stage_v[srow, sl] = buf_v[half + s, ov[s], sl] * SCALE

        fire(0, sem_a)

        def body(t, carry):
            fire(2 * t + 1, sem_b)
            drain(sem_a)
            select(2 * t)

            @pl.when(t < n_batch // 2 - 1)
            def _():
                fire(2 * t + 2, sem_a)

            drain(sem_b)
            select(2 * t + 1)

            @pl.when((t & 3) == 3)
            def _():
                pltpu.sync_copy(
                    stage_v, out_hbm.at[pl.ds(base + (t >> 2) * och, och)]
                )

            return carry

        lax.fori_loop(0, n_batch // 2, body, 0)

    return k(x3, table3)


def kernel(x, weight):
    b0, b1 = x.shape
    b_total = b0 * b1
    b_per_w = b_total // _NW
    x3 = x.astype(jnp.int32).reshape(_NW, RPG, b_per_w // RPG)
    table3 = weight.reshape(weight.shape[0] // RPG, RPG, D_MODEL)
    out = _emb_lookup(x3, table3, b_total, D_MODEL)
    return out.reshape(b0, b1, D_MODEL)


# speedup vs baseline: 2.3661x; 1.0252x over previous
import functools
import math

import jax
import jax.numpy as jnp
from jax import lax
from jax.experimental import pallas as pl
from jax.experimental.pallas import tpu as pltpu
from jax.experimental.pallas import tpu_sc as plsc

D_MODEL = 64
SCALE = math.sqrt(D_MODEL)
RPG = 8
NGRP = 4

_info = plsc.get_sparse_core_info()
_NC, _NS, _L = _info.num_cores, _info.num_subcores, _info.num_lanes
_NW = _NC * _NS


@functools.partial(jax.jit, static_argnames=("out_shape", "d"))
def _emb_lookup(x3, table3, out_shape, d):
    b_total = out_shape[0] * out_shape[1]
    b_per_w = b_total // _NW
    n_batch = b_per_w // _L
    och = 128
    mesh = plsc.VectorSubcoreMesh(core_axis_name="c", subcore_axis_name="s")

    @functools.partial(
        pl.kernel,
        mesh=mesh,
        out_type=jax.ShapeDtypeStruct((*out_shape, d), jnp.float32),
        scratch_types=[
            pltpu.VMEM((RPG, b_per_w // RPG), jnp.int32),
            pltpu.VMEM((NGRP * _L, RPG, d), jnp.float32),
            pltpu.VMEM((och, d), jnp.float32),
            [pltpu.SemaphoreType.DMA] * NGRP,
        ],
    )
    def k(idx_hbm, tab_hbm, out_hbm, idx_v, buf_v, stage_v, sems):
        wid = lax.axis_index("s") * _NC + lax.axis_index("c")
        base = wid * b_per_w
        out2 = out_hbm.reshape(b_total, d)
        pltpu.sync_copy(idx_hbm.at[wid], idx_v)
        ncol = b_per_w // RPG

        def batch_vec(bb):
            j = (bb * _L) // ncol
            col = (bb * _L) % ncol
            return idx_v[j, pl.ds(col, _L)]

        def fire(bb, u):
            gv = batch_vec(bb) >> 3
            for s in range(_L):
                pltpu.async_copy(tab_hbm.at[gv[s]], buf_v.at[u * _L + s], sems[u])

        def drain(u):
            for _ in range(_L):
                pltpu.make_async_copy(tab_hbm.at[0], buf_v.at[0], sems[u]).wait()

        def select(bb, u):
            ov = batch_vec(bb) & 7
            for s in range(_L):
                srow = ((bb * _L) % och) + s
                for kk in range(d // _L):
                    sl = pl.ds(kk * _L, _L)
                    stage_v[srow, sl] = buf_v[u * _L + s, ov[s], sl] * SCALE

        for u in range(NGRP):
            fire(u, u)

        def body(t, carry):
            for u in range(NGRP):
                bb = NGRP * t + u
                drain(u)
                select(bb, u)

                @pl.when(t < n_batch // NGRP - 1)
                def _():
                    fire(bb + NGRP, u)

            @pl.when((t & 1) == 1)
            def _():
                pltpu.sync_copy(
                    stage_v, out2.at[pl.ds(base + (t >> 1) * och, och)]
                )

            return carry

        lax.fori_loop(0, n_batch // NGRP, body, 0)

    return k(x3, table3)


def kernel(x, weight):
    b0, b1 = x.shape
    b_total = b0 * b1
    b_per_w = b_total // _NW
    x3 = x.astype(jnp.int32).reshape(_NW, RPG, b_per_w // RPG)
    table3 = weight.reshape(weight.shape[0] // RPG, RPG, D_MODEL)
    return _emb_lookup(x3, table3, (b0, b1), D_MODEL)


# speedup vs baseline: 2.3712x; 1.0021x over previous
import functools
import math

import jax
import jax.numpy as jnp
from jax import lax
from jax.experimental import pallas as pl
from jax.experimental.pallas import tpu as pltpu
from jax.experimental.pallas import tpu_sc as plsc

D_MODEL = 64
SCALE = math.sqrt(D_MODEL)
RPG = 8
NGRP = 4

_info = plsc.get_sparse_core_info()
_NC, _NS, _L = _info.num_cores, _info.num_subcores, _info.num_lanes
_NW = _NC * _NS


@functools.partial(jax.jit, static_argnames=("out_shape", "d"))
def _emb_lookup(x3, table3, out_shape, d):
    b_total = out_shape[0] * out_shape[1]
    b_per_w = b_total // _NW
    n_batch = b_per_w // _L
    och = 128
    mesh = plsc.VectorSubcoreMesh(core_axis_name="c", subcore_axis_name="s")

    @functools.partial(
        pl.kernel,
        mesh=mesh,
        out_type=jax.ShapeDtypeStruct((*out_shape, d), jnp.float32),
        scratch_types=[
            pltpu.VMEM((RPG, b_per_w // RPG), jnp.int32),
            pltpu.VMEM((NGRP * _L, RPG, d), jnp.float32),
            pltpu.VMEM((och, d), jnp.float32),
            [pltpu.SemaphoreType.DMA] * NGRP,
        ],
    )
    def k(idx_hbm, tab_hbm, out_hbm, idx_v, buf_v, stage_v, sems):
        wid = lax.axis_index("s") * _NC + lax.axis_index("c")
        base = wid * b_per_w
        out2 = out_hbm.reshape(b_total, d)
        pltpu.sync_copy(idx_hbm.at[wid], idx_v)
        ncol = b_per_w // RPG

        def batch_vec(bb):
            j = (bb * _L) // ncol
            col = (bb * _L) % ncol
            return idx_v[j, pl.ds(col, _L)]

        def fire(bb, u):
            gv = batch_vec(bb) >> 3
            for s in range(_L):
                pltpu.async_copy(tab_hbm.at[gv[s]], buf_v.at[u * _L + s], sems[u])

        def drain(u):
            pltpu.make_async_copy(
                tab_hbm.at[pl.ds(0, _L)], buf_v.at[pl.ds(0, _L)], sems[u]
            ).wait()

        def select(bb, u):
            ov = batch_vec(bb) & 7
            for s in range(_L):
                srow = ((bb * _L) % och) + s
                for kk in range(d // _L):
                    sl = pl.ds(kk * _L, _L)
                    stage_v[srow, sl] = buf_v[u * _L + s, ov[s], sl] * SCALE

        for u in range(NGRP):
            fire(u, u)

        def body(t, carry):
            for u in range(NGRP):
                bb = NGRP * t + u
                drain(u)
                select(bb, u)

                @pl.when(t < n_batch // NGRP - 1)
                def _():
                    fire(bb + NGRP, u)

            @pl.when((t & 1) == 1)
            def _():
                pltpu.sync_copy(
                    stage_v, out2.at[pl.ds(base + (t >> 1) * och, och)]
                )

            return carry

        lax.fori_loop(0, n_batch // NGRP, body, 0)

    return k(x3, table3)


def kernel(x, weight):
    b0, b1 = x.shape
    b_total = b0 * b1
    b_per_w = b_total // _NW
    x3 = x.astype(jnp.int32).reshape(_NW, RPG, b_per_w // RPG)
    table3 = weight.reshape(weight.shape[0] // RPG, RPG, D_MODEL)
    return _emb_lookup(x3, table3, (b0, b1), D_MODEL)
